# baseline (speedup 1.0000x reference)
import functools

import jax
import jax.numpy as jnp
from jax import lax
from jax.experimental import pallas as pl
from jax.experimental.pallas import tpu as pltpu
from jax.experimental.pallas import tpu_sc as plsc

_N = 128
_C = 32768
_L = 16
_NW = 32
_RPW = _N // _NW
_NCH = _C // _L

_mesh = plsc.VectorSubcoreMesh(core_axis_name="c", subcore_axis_name="s")


@functools.partial(
    pl.kernel,
    out_type=(jax.ShapeDtypeStruct((_N, _L), jnp.float32),
              jax.ShapeDtypeStruct((_N, _L), jnp.int32)),
    mesh=_mesh,
    scratch_types=[
        pltpu.VMEM((2, _C), jnp.float32),
        pltpu.VMEM((_RPW, _L), jnp.float32),
        pltpu.VMEM((_RPW, _L), jnp.int32),
        pltpu.SemaphoreType.DMA((2,)),
        pltpu.SemaphoreType.DMA((_RPW,)),
        pltpu.SemaphoreType.DMA((_RPW,)),
    ],
)
def _sc_argmax(x_hbm, val_hbm, idx_hbm, row_buf, val_buf, idx_buf,
               in_sem, val_sem, idx_sem):
    wid = lax.axis_index("s") * 2 + lax.axis_index("c")
    base = wid * _RPW

    def get(r, slot):
        return pltpu.make_async_copy(
            x_hbm.at[base + r], row_buf.at[slot], in_sem.at[slot])

    get(0, 0).start()
    lane = lax.broadcasted_iota(jnp.int32, (_L,), 0)
    for r in range(_RPW):
        if r + 1 < _RPW:
            get(r + 1, (r + 1) % 2).start()
        get(r, r % 2).wait()

        def body(j, carry, _slot=r % 2):
            bv, bi = carry
            v = row_buf[_slot, pl.ds(j * _L, _L)]
            take = v > bv
            return (jnp.where(take, v, bv),
                    jnp.where(take, j * _L + lane, bi))

        bv, bi = lax.fori_loop(
            0, _NCH, body,
            (jnp.full((_L,), -jnp.inf, jnp.float32),
             jnp.zeros((_L,), jnp.int32)))
        val_buf[r] = bv
        idx_buf[r] = bi
        pltpu.make_async_copy(
            val_buf.at[r], val_hbm.at[base + r], val_sem.at[r]).start()
        pltpu.make_async_copy(
            idx_buf.at[r], idx_hbm.at[base + r], idx_sem.at[r]).start()
    for r in range(_RPW):
        pltpu.make_async_copy(
            val_buf.at[r], val_hbm.at[base + r], val_sem.at[r]).wait()
        pltpu.make_async_copy(
            idx_buf.at[r], idx_hbm.at[base + r], idx_sem.at[r]).wait()


_RB = 16
_NK = _N // _RB
_NS = 4


def _onehot_body(val_ref, idx_ref, o_hbm, buf, iota_buf, out_sem):
    iota_buf[...] = lax.broadcasted_iota(jnp.int32, (_RB, _C), 1)

    def put(k, slot):
        return pltpu.make_async_copy(
            buf.at[slot], o_hbm.at[pl.ds(k * _RB, _RB), :], out_sem.at[slot])

    for k in range(_NK):
        slot = k % _NS
        if k >= _NS:
            put(k - _NS, slot).wait()
        bv = val_ref[pl.ds(k * _RB, _RB), :]
        bi = idx_ref[pl.ds(k * _RB, _RB), :]
        m = jnp.max(bv, axis=1, keepdims=True)
        idxv = jnp.min(jnp.where(bv == m, bi, _C), axis=1, keepdims=True)
        buf[slot] = (iota_buf[...] == idxv).astype(jnp.float32)
        put(k, slot).start()
    for k in range(max(_NK - _NS, 0), _NK):
        put(k, k % _NS).wait()


def _onehot(val, idx):
    return pl.pallas_call(
        _onehot_body,
        in_specs=[pl.BlockSpec((_N, _L), lambda: (0, 0)),
                  pl.BlockSpec((_N, _L), lambda: (0, 0))],
        out_specs=pl.BlockSpec(memory_space=pl.MemorySpace.ANY),
        out_shape=jax.ShapeDtypeStruct((_N, _C), jnp.float32),
        scratch_shapes=[
            pltpu.VMEM((_NS, _RB, _C), jnp.float32),
            pltpu.VMEM((_RB, _C), jnp.int32),
            pltpu.SemaphoreType.DMA((_NS,)),
        ],
    )(val, idx)


@jax.jit
def kernel(x):
    val, idx = _sc_argmax(x)
    return _onehot(val, idx)


# speedup vs baseline: 1.5584x; 1.5584x over previous
import functools

import jax
import jax.numpy as jnp
from jax import lax
from jax.experimental import pallas as pl
from jax.experimental.pallas import tpu as pltpu
from jax.experimental.pallas import tpu_sc as plsc

_N = 128
_C = 32768
_L = 16
_NW = 32
_RPW = _N // _NW
_NCH = _C // _L
_U = 8

_mesh = plsc.VectorSubcoreMesh(core_axis_name="c", subcore_axis_name="s")


@functools.partial(
    pl.kernel,
    out_type=(jax.ShapeDtypeStruct((_N, _L), jnp.float32),
              jax.ShapeDtypeStruct((_N, _L), jnp.int32)),
    mesh=_mesh,
    scratch_types=[
        pltpu.VMEM((2, _C), jnp.float32),
        pltpu.VMEM((_RPW, _L), jnp.float32),
        pltpu.VMEM((_RPW, _L), jnp.int32),
        pltpu.SemaphoreType.DMA((2,)),
        pltpu.SemaphoreType.DMA((_RPW,)),
        pltpu.SemaphoreType.DMA((_RPW,)),
    ],
)
def _sc_argmax(x_hbm, val_hbm, idx_hbm, row_buf, val_buf, idx_buf,
               in_sem, val_sem, idx_sem):
    wid = lax.axis_index("s") * 2 + lax.axis_index("c")
    base = wid * _RPW

    def get(r, slot):
        return pltpu.make_async_copy(
            x_hbm.at[base + r], row_buf.at[slot], in_sem.at[slot])

    get(0, 0).start()
    lane = lax.broadcasted_iota(jnp.int32, (_L,), 0)
    for r in range(_RPW):
        if r + 1 < _RPW:
            get(r + 1, (r + 1) % 2).start()
        get(r, r % 2).wait()

        def body(j, carry, _slot=r % 2):
            out = []
            for u in range(_U):
                bv, bi = carry[2 * u], carry[2 * u + 1]
                v = row_buf[_slot, pl.ds((j * _U + u) * _L, _L)]
                take = v > bv
                out.append(jnp.where(take, v, bv))
                out.append(jnp.where(take, (j * _U + u) * _L + lane, bi))
            return tuple(out)

        init = []
        for u in range(_U):
            init.append(jnp.full((_L,), -jnp.inf, jnp.float32))
            init.append(jnp.zeros((_L,), jnp.int32))
        acc = lax.fori_loop(0, _NCH // _U, body, tuple(init))
        bv, bi = acc[0], acc[1]
        for u in range(1, _U):
            v2, i2 = acc[2 * u], acc[2 * u + 1]
            take = (v2 > bv) | ((v2 == bv) & (i2 < bi))
            bv = jnp.where(take, v2, bv)
            bi = jnp.where(take, i2, bi)
        val_buf[r] = bv
        idx_buf[r] = bi
        pltpu.make_async_copy(
            val_buf.at[r], val_hbm.at[base + r], val_sem.at[r]).start()
        pltpu.make_async_copy(
            idx_buf.at[r], idx_hbm.at[base + r], idx_sem.at[r]).start()
    for r in range(_RPW):
        pltpu.make_async_copy(
            val_buf.at[r], val_hbm.at[base + r], val_sem.at[r]).wait()
        pltpu.make_async_copy(
            idx_buf.at[r], idx_hbm.at[base + r], idx_sem.at[r]).wait()


_RB = 16
_NK = _N // _RB
_NS = 4


def _onehot_body(val_ref, idx_ref, o_hbm, buf, iota_buf, out_sem):
    iota_buf[...] = lax.broadcasted_iota(jnp.int32, (_RB, _C), 1)

    def put(k, slot):
        return pltpu.make_async_copy(
            buf.at[slot], o_hbm.at[pl.ds(k * _RB, _RB), :], out_sem.at[slot])

    for k in range(_NK):
        slot = k % _NS
        if k >= _NS:
            put(k - _NS, slot).wait()
        bv = val_ref[pl.ds(k * _RB, _RB), :]
        bi = idx_ref[pl.ds(k * _RB, _RB), :]
        m = jnp.max(bv, axis=1, keepdims=True)
        idxv = jnp.min(jnp.where(bv == m, bi, _C), axis=1, keepdims=True)
        buf[slot] = (iota_buf[...] == idxv).astype(jnp.float32)
        put(k, slot).start()
    for k in range(max(_NK - _NS, 0), _NK):
        put(k, k % _NS).wait()


def _onehot(val, idx):
    return pl.pallas_call(
        _onehot_body,
        in_specs=[pl.BlockSpec((_N, _L), lambda: (0, 0)),
                  pl.BlockSpec((_N, _L), lambda: (0, 0))],
        out_specs=pl.BlockSpec(memory_space=pl.MemorySpace.ANY),
        out_shape=jax.ShapeDtypeStruct((_N, _C), jnp.float32),
        scratch_shapes=[
            pltpu.VMEM((_NS, _RB, _C), jnp.float32),
            pltpu.VMEM((_RB, _C), jnp.int32),
            pltpu.SemaphoreType.DMA((_NS,)),
        ],
    )(val, idx)


@jax.jit
def kernel(x):
    val, idx = _sc_argmax(x)
    return _onehot(val, idx)


# speedup vs baseline: 1.9728x; 1.2659x over previous
import functools

import jax
import jax.numpy as jnp
from jax import lax
from jax.experimental import pallas as pl
from jax.experimental.pallas import tpu as pltpu
from jax.experimental.pallas import tpu_sc as plsc

_N = 128
_C = 32768
_SPLIT = 96
_L = 16
_NW = 32
_NCH = _C // _L
_U = 8

_RB = 16
_NS = 4
_NK1 = _SPLIT // _RB
_NK2 = (_N - _SPLIT) // _RB


def _tc_main_body(x_hbm, o_hbm, in_buf, out_buf, in_sem, out_sem):
    def get(k, slot):
        return pltpu.make_async_copy(
            x_hbm.at[pl.ds(k * _RB, _RB), :], in_buf.at[slot], in_sem.at[slot])

    def put(k, slot):
        return pltpu.make_async_copy(
            out_buf.at[slot], o_hbm.at[pl.ds(k * _RB, _RB), :], out_sem.at[slot])

    for k in range(_NS - 1):
        get(k, k % _NS).start()
    for k in range(_NK1):
        slot = k % _NS
        if k + _NS - 1 < _NK1:
            get(k + _NS - 1, (k + _NS - 1) % _NS).start()
        get(k, slot).wait()
        xb = in_buf[slot]
        idx = jnp.argmax(xb, axis=1)
        ii = lax.broadcasted_iota(jnp.int32, (_RB, _C), 1)
        if k >= _NS:
            put(k - _NS, slot).wait()
        out_buf[slot] = (ii == idx[:, None]).astype(jnp.float32)
        put(k, slot).start()
    for k in range(max(_NK1 - _NS, 0), _NK1):
        put(k, k % _NS).wait()


def _tc_main(x):
    return pl.pallas_call(
        _tc_main_body,
        in_specs=[pl.BlockSpec(memory_space=pl.MemorySpace.ANY)],
        out_specs=pl.BlockSpec(memory_space=pl.MemorySpace.ANY),
        out_shape=jax.ShapeDtypeStruct((_N, _C), jnp.float32),
        scratch_shapes=[
            pltpu.VMEM((_NS, _RB, _C), jnp.float32),
            pltpu.VMEM((_NS, _RB, _C), jnp.float32),
            pltpu.SemaphoreType.DMA((_NS,)),
            pltpu.SemaphoreType.DMA((_NS,)),
        ],
    )(x)


_mesh = plsc.VectorSubcoreMesh(core_axis_name="c", subcore_axis_name="s")


@functools.partial(
    pl.kernel,
    out_type=(jax.ShapeDtypeStruct((_NW, _L), jnp.float32),
              jax.ShapeDtypeStruct((_NW, _L), jnp.int32)),
    mesh=_mesh,
    scratch_types=[
        pltpu.VMEM((_C,), jnp.float32),
        pltpu.VMEM((_L,), jnp.float32),
        pltpu.VMEM((_L,), jnp.int32),
        pltpu.SemaphoreType.DMA,
        pltpu.SemaphoreType.DMA,
        pltpu.SemaphoreType.DMA,
    ],
)
def _sc_argmax(x_hbm, val_hbm, idx_hbm, row_buf, val_buf, idx_buf,
               in_sem, val_sem, idx_sem):
    wid = lax.axis_index("s") * 2 + lax.axis_index("c")
    row = _SPLIT + wid
    pltpu.make_async_copy(x_hbm.at[row], row_buf, in_sem).start()
    pltpu.make_async_copy(x_hbm.at[row], row_buf, in_sem).wait()
    lane = lax.broadcasted_iota(jnp.int32, (_L,), 0)

    def body(j, carry):
        out = []
        for u in range(_U):
            bv, bi = carry[2 * u], carry[2 * u + 1]
            v = row_buf[pl.ds((j * _U + u) * _L, _L)]
            take = v > bv
            out.append(jnp.where(take, v, bv))
            out.append(jnp.where(take, (j * _U + u) * _L + lane, bi))
        return tuple(out)

    init = []
    for u in range(_U):
        init.append(jnp.full((_L,), -jnp.inf, jnp.float32))
        init.append(jnp.zeros((_L,), jnp.int32))
    acc = lax.fori_loop(0, _NCH // _U, body, tuple(init))
    bv, bi = acc[0], acc[1]
    for u in range(1, _U):
        v2, i2 = acc[2 * u], acc[2 * u + 1]
        take = (v2 > bv) | ((v2 == bv) & (i2 < bi))
        bv = jnp.where(take, v2, bv)
        bi = jnp.where(take, i2, bi)
    val_buf[...] = bv
    idx_buf[...] = bi
    pltpu.make_async_copy(val_buf, val_hbm.at[wid], val_sem).start()
    pltpu.make_async_copy(idx_buf, idx_hbm.at[wid], idx_sem).start()
    pltpu.make_async_copy(val_buf, val_hbm.at[wid], val_sem).wait()
    pltpu.make_async_copy(idx_buf, idx_hbm.at[wid], idx_sem).wait()


def _tc_tail_body(big_ref, val_ref, idx_ref, o_hbm, buf, out_sem):
    del big_ref

    def put(k, slot):
        return pltpu.make_async_copy(
            buf.at[slot],
            o_hbm.at[pl.ds(_SPLIT + k * _RB, _RB), :],
            out_sem.at[slot])

    ii = lax.broadcasted_iota(jnp.int32, (_RB, _C), 1)
    for k in range(_NK2):
        bv = val_ref[pl.ds(k * _RB, _RB), :]
        bi = idx_ref[pl.ds(k * _RB, _RB), :]
        m = jnp.max(bv, axis=1, keepdims=True)
        idxv = jnp.min(jnp.where(bv == m, bi, _C), axis=1, keepdims=True)
        buf[k] = (ii == idxv).astype(jnp.float32)
        put(k, k).start()
    for k in range(_NK2):
        put(k, k).wait()


def _tc_tail(big, val, idx):
    return pl.pallas_call(
        _tc_tail_body,
        in_specs=[pl.BlockSpec(memory_space=pl.MemorySpace.ANY),
                  pl.BlockSpec((_NW, _L), lambda: (0, 0)),
                  pl.BlockSpec((_NW, _L), lambda: (0, 0))],
        out_specs=pl.BlockSpec(memory_space=pl.MemorySpace.ANY),
        out_shape=jax.ShapeDtypeStruct((_N, _C), jnp.float32),
        input_output_aliases={0: 0},
        scratch_shapes=[
            pltpu.VMEM((_NK2, _RB, _C), jnp.float32),
            pltpu.SemaphoreType.DMA((_NK2,)),
        ],
    )(big, val, idx)


@jax.jit
def kernel(x):
    big = _tc_main(x)
    val, idx = _sc_argmax(x)
    return _tc_tail(big, val, idx)


# speedup vs baseline: 5.4917x; 2.7836x over previous
import jax
import jax.numpy as jnp
from jax.experimental import pallas as pl
from jax.experimental.pallas import tpu as pltpu

_N = 128
_C = 32768
_RB = 16
_NK = _N // _RB
_NS = 8


def _ste_body(x_hbm, o_hbm, in_buf, out_buf, in_sem, out_sem):
    def get_copy(k, slot):
        return pltpu.make_async_copy(
            x_hbm.at[pl.ds(k * _RB, _RB), :], in_buf.at[slot], in_sem.at[slot])

    def put_copy(k, slot):
        return pltpu.make_async_copy(
            out_buf.at[slot], o_hbm.at[pl.ds(k * _RB, _RB), :], out_sem.at[slot])

    for k in range(_NS - 1):
        get_copy(k, k % _NS).start()
    for k in range(_NK):
        slot = k % _NS
        if k + _NS - 1 < _NK:
            get_copy(k + _NS - 1, (k + _NS - 1) % _NS).start()
        get_copy(k, slot).wait()
        xb = in_buf[slot]
        idx = jnp.argmax(xb, axis=1)
        ii = jax.lax.broadcasted_iota(jnp.int32, (_RB, _C), 1)
        if k >= _NS:
            put_copy(k - _NS, slot).wait()
        out_buf[slot] = (ii == idx[:, None]).astype(jnp.float32)
        put_copy(k, slot).start()
    for k in range(max(_NK - _NS, 0), _NK):
        put_copy(k, k % _NS).wait()


@jax.jit
def kernel(x):
    return pl.pallas_call(
        _ste_body,
        in_specs=[pl.BlockSpec(memory_space=pl.MemorySpace.ANY)],
        out_specs=pl.BlockSpec(memory_space=pl.MemorySpace.ANY),
        out_shape=jax.ShapeDtypeStruct((_N, _C), jnp.float32),
        scratch_shapes=[
            pltpu.VMEM((_NS, _RB, _C), jnp.float32),
            pltpu.VMEM((_NS, _RB, _C), jnp.float32),
            pltpu.SemaphoreType.DMA((_NS,)),
            pltpu.SemaphoreType.DMA((_NS,)),
        ],
    )(x)


# speedup vs baseline: 5.6232x; 1.0239x over previous
import jax
import jax.numpy as jnp
from jax.experimental import pallas as pl
from jax.experimental.pallas import tpu as pltpu

_N = 128
_C = 32768
_RB = 32
_NK = _N // _RB
_NS = 4


def _ste_body(x_hbm, o_hbm, in_buf, out_buf, in_sem, out_sem):
    def get_copy(k, slot):
        return pltpu.make_async_copy(
            x_hbm.at[pl.ds(k * _RB, _RB), :], in_buf.at[slot], in_sem.at[slot])

    def put_copy(k, slot):
        return pltpu.make_async_copy(
            out_buf.at[slot], o_hbm.at[pl.ds(k * _RB, _RB), :], out_sem.at[slot])

    for k in range(_NS - 1):
        get_copy(k, k % _NS).start()
    for k in range(_NK):
        slot = k % _NS
        if k + _NS - 1 < _NK:
            get_copy(k + _NS - 1, (k + _NS - 1) % _NS).start()
        get_copy(k, slot).wait()
        xb = in_buf[slot]
        idx = jnp.argmax(xb, axis=1)
        ii = jax.lax.broadcasted_iota(jnp.int32, (_RB, _C), 1)
        if k >= _NS:
            put_copy(k - _NS, slot).wait()
        out_buf[slot] = (ii == idx[:, None]).astype(jnp.float32)
        put_copy(k, slot).start()
    for k in range(max(_NK - _NS, 0), _NK):
        put_copy(k, k % _NS).wait()


@jax.jit
def kernel(x):
    return pl.pallas_call(
        _ste_body,
        in_specs=[pl.BlockSpec(memory_space=pl.MemorySpace.ANY)],
        out_specs=pl.BlockSpec(memory_space=pl.MemorySpace.ANY),
        out_shape=jax.ShapeDtypeStruct((_N, _C), jnp.float32),
        scratch_shapes=[
            pltpu.VMEM((_NS, _RB, _C), jnp.float32),
            pltpu.VMEM((_NS, _RB, _C), jnp.float32),
            pltpu.SemaphoreType.DMA((_NS,)),
            pltpu.SemaphoreType.DMA((_NS,)),
        ],
    )(x)
